# Initial kernel scaffold; baseline (speedup 1.0000x reference)
import jax
import jax.numpy as jnp
from jax.experimental import pallas as pl


def kernel(game_state, A, obj_table, node_table, W0, W1, W2, Wf, bf):
    raise NotImplementedError("write your pallas kernel here")


# speedup vs baseline: 28.4606x; 28.4606x over previous
import dataclasses
import functools

import jax
import jax.numpy as jnp
from jax import lax
from jax.experimental import pallas as pl
from jax.experimental.pallas import tpu as pltpu
from jax.experimental.pallas import tpu_sc as plsc

N_NODES = 16
NUM_TYPES = 1000
EMB = 4
B, H, W = 4096, 32, 32
HW = H * W
OUT_ROW = EMB * HW
LANES = 16

NUM_CORES = 2
NUM_SUBCORES = 16
NW = NUM_CORES * NUM_SUBCORES
B_PER_W = B // NW
CHUNK_B = 8
NCHUNK = B_PER_W // CHUNK_B
CHUNK_IDX = CHUNK_B * HW
CHUNK_OUT = CHUNK_B * OUT_ROW


def _gather_body(gs_hbm, tab_hbm, out_hbm, tab_v, idx_v0, idx_v1,
                 out_v0, out_v1, sem_in0, sem_in1, sem_out0, sem_out1):
    wid = lax.axis_index("s") * NUM_CORES + lax.axis_index("c")
    base = wid * B_PER_W
    idx_bufs = (idx_v0, idx_v1)
    out_bufs = (out_v0, out_v1)
    sems_in = (sem_in0, sem_in1)
    sems_out = (sem_out0, sem_out1)

    pltpu.sync_copy(tab_hbm, tab_v)

    def start_in(c, buf):
        off = (base + c * CHUNK_B) * HW
        return pltpu.async_copy(gs_hbm.at[pl.ds(off, CHUNK_IDX)],
                                idx_bufs[buf], sems_in[buf])

    def start_out(c, buf):
        off = (base + c * CHUNK_B) * OUT_ROW
        return pltpu.async_copy(out_bufs[buf],
                                out_hbm.at[pl.ds(off, CHUNK_OUT)],
                                sems_out[buf])

    def compute(buf):
        idxb = idx_bufs[buf]
        outb = out_bufs[buf]

        @pl.loop(0, CHUNK_B)
        def _(b):
            ibase = b * HW
            obase = b * OUT_ROW

            @pl.loop(0, HW // LANES)
            def _(g):
                ii = idxb[pl.ds(ibase + g * LANES, LANES)]
                i4 = ii * 4
                o = obase + g * LANES
                outb[pl.ds(o, LANES)] = plsc.load_gather(tab_v, [i4])
                outb[pl.ds(o + HW, LANES)] = plsc.load_gather(tab_v, [i4 + 1])
                outb[pl.ds(o + 2 * HW, LANES)] = plsc.load_gather(
                    tab_v, [i4 + 2])
                outb[pl.ds(o + 3 * HW, LANES)] = plsc.load_gather(
                    tab_v, [i4 + 3])

    copies_out = [None, None]
    copy_in = [None, None]
    copy_in[0] = start_in(0, 0)
    for c in range(NCHUNK):
        buf = c & 1
        if c + 1 < NCHUNK:
            copy_in[1 - buf] = start_in(c + 1, 1 - buf)
        copy_in[buf].wait()
        if copies_out[buf] is not None:
            copies_out[buf].wait()
        compute(buf)
        copies_out[buf] = start_out(c, buf)
    copies_out[0].wait()
    copies_out[1].wait()


@jax.jit
def _sc_gather(gs_flat, tab_flat):
    mesh = plsc.VectorSubcoreMesh(core_axis_name="c", subcore_axis_name="s")
    cp = pltpu.CompilerParams()
    if "needs_layout_passes" in pltpu.CompilerParams.__dataclass_fields__:
        cp = dataclasses.replace(cp, needs_layout_passes=False)
    k = pl.kernel(
        _gather_body,
        out_type=jax.ShapeDtypeStruct((B * OUT_ROW,), jnp.float32),
        mesh=mesh,
        scratch_types=[
            pltpu.VMEM((NUM_TYPES * EMB,), jnp.float32),
            pltpu.VMEM((CHUNK_IDX,), jnp.int32),
            pltpu.VMEM((CHUNK_IDX,), jnp.int32),
            pltpu.VMEM((CHUNK_OUT,), jnp.float32),
            pltpu.VMEM((CHUNK_OUT,), jnp.float32),
            pltpu.SemaphoreType.DMA,
            pltpu.SemaphoreType.DMA,
            pltpu.SemaphoreType.DMA,
            pltpu.SemaphoreType.DMA,
        ],
        compiler_params=cp,
    )
    return k(gs_flat, tab_flat)


def _gcn_body(a_ref, nt_ref, w0_ref, w1_ref, w2_ref, wf_ref, bf_ref, out_ref):
    hi = lax.Precision.HIGHEST
    nt = ((1,), (1,)), ((), ())
    a = a_ref[...]
    x = jnp.dot(a, nt_ref[...], precision=hi)
    x = jnp.maximum(lax.dot_general(x, w0_ref[...], nt, precision=hi), 0.0)
    x = jnp.dot(a, x, precision=hi)
    x = jnp.maximum(lax.dot_general(x, w1_ref[...], nt, precision=hi), 0.0)
    x = jnp.dot(a, x, precision=hi)
    x = jnp.maximum(lax.dot_general(x, w2_ref[...], nt, precision=hi), 0.0)
    out_ref[...] = lax.dot_general(x, wf_ref[...], nt, precision=hi) \
        + bf_ref[...]


@jax.jit
def _gcn(A, node_table, W0, W1, W2, Wf, bf):
    return pl.pallas_call(
        _gcn_body,
        out_shape=jax.ShapeDtypeStruct((N_NODES, EMB), jnp.float32),
    )(A, node_table, W0, W1, W2, Wf, bf.reshape(1, EMB))


def kernel(game_state, A, obj_table, node_table, W0, W1, W2, Wf, bf):
    gs_flat = game_state.reshape(-1)
    tab_flat = obj_table.reshape(-1)
    out_flat = _sc_gather(gs_flat, tab_flat)
    out = out_flat.reshape(B, EMB, H, W)
    node_embeddings = _gcn(A, node_table, W0, W1, W2, Wf, bf)
    return (out, node_embeddings)


# speedup vs baseline: 40.5762x; 1.4257x over previous
import dataclasses
import functools

import jax
import jax.numpy as jnp
from jax import lax
from jax.experimental import pallas as pl
from jax.experimental.pallas import tpu as pltpu
from jax.experimental.pallas import tpu_sc as plsc

N_NODES = 16
NUM_TYPES = 1000
EMB = 4
B, H, W = 4096, 32, 32
HW = H * W
OUT_ROW = EMB * HW
LANES = 16

NUM_CORES = 2
NUM_SUBCORES = 16
NW = NUM_CORES * NUM_SUBCORES
B_PER_W = B // NW
CHUNK_B = 8
NCHUNK = B_PER_W // CHUNK_B
CHUNK_IDX = CHUNK_B * HW
CHUNK_OUT = CHUNK_B * OUT_ROW


def _gather_body(gs_hbm, tab_hbm, out_hbm, tab_v, idx_v0, idx_v1,
                 out_v0, out_v1, sem_in0, sem_in1, sem_out0, sem_out1):
    wid = lax.axis_index("s") * NUM_CORES + lax.axis_index("c")
    base = wid * B_PER_W
    idx_bufs = (idx_v0, idx_v1)
    out_bufs = (out_v0, out_v1)
    sems_in = (sem_in0, sem_in1)
    sems_out = (sem_out0, sem_out1)

    pltpu.sync_copy(tab_hbm, tab_v)

    def start_in(c, buf):
        off = (base + c * CHUNK_B) * HW
        return pltpu.async_copy(gs_hbm.at[pl.ds(off, CHUNK_IDX)],
                                idx_bufs[buf], sems_in[buf])

    def start_out(c, buf):
        off = (base + c * CHUNK_B) * OUT_ROW
        return pltpu.async_copy(out_bufs[buf],
                                out_hbm.at[pl.ds(off, CHUNK_OUT)],
                                sems_out[buf])

    def compute(buf):
        idxb = idx_bufs[buf]
        outb = out_bufs[buf]
        groups_per_row = HW // LANES

        @plsc.parallel_loop(0, CHUNK_IDX // LANES, unroll=8)
        def _(j):
            ii = idxb[pl.ds(j * LANES, LANES)]
            i4 = ii * 4
            o = j * LANES + (j // groups_per_row) * (OUT_ROW - HW)
            outb[pl.ds(o, LANES)] = plsc.load_gather(tab_v, [i4])
            outb[pl.ds(o + HW, LANES)] = plsc.load_gather(tab_v, [i4 + 1])
            outb[pl.ds(o + 2 * HW, LANES)] = plsc.load_gather(
                tab_v, [i4 + 2])
            outb[pl.ds(o + 3 * HW, LANES)] = plsc.load_gather(
                tab_v, [i4 + 3])

    copies_out = [None, None]
    copy_in = [None, None]
    copy_in[0] = start_in(0, 0)
    for c in range(NCHUNK):
        buf = c & 1
        if c + 1 < NCHUNK:
            copy_in[1 - buf] = start_in(c + 1, 1 - buf)
        copy_in[buf].wait()
        if copies_out[buf] is not None:
            copies_out[buf].wait()
        compute(buf)
        copies_out[buf] = start_out(c, buf)
    copies_out[0].wait()
    copies_out[1].wait()


@jax.jit
def _sc_gather(gs_flat, tab_flat):
    mesh = plsc.VectorSubcoreMesh(core_axis_name="c", subcore_axis_name="s")
    cp = pltpu.CompilerParams()
    if "needs_layout_passes" in pltpu.CompilerParams.__dataclass_fields__:
        cp = dataclasses.replace(cp, needs_layout_passes=False)
    k = pl.kernel(
        _gather_body,
        out_type=jax.ShapeDtypeStruct((B * OUT_ROW,), jnp.float32),
        mesh=mesh,
        scratch_types=[
            pltpu.VMEM((NUM_TYPES * EMB,), jnp.float32),
            pltpu.VMEM((CHUNK_IDX,), jnp.int32),
            pltpu.VMEM((CHUNK_IDX,), jnp.int32),
            pltpu.VMEM((CHUNK_OUT,), jnp.float32),
            pltpu.VMEM((CHUNK_OUT,), jnp.float32),
            pltpu.SemaphoreType.DMA,
            pltpu.SemaphoreType.DMA,
            pltpu.SemaphoreType.DMA,
            pltpu.SemaphoreType.DMA,
        ],
        compiler_params=cp,
    )
    return k(gs_flat, tab_flat)


def _gcn_body(a_ref, nt_ref, w0_ref, w1_ref, w2_ref, wf_ref, bf_ref, out_ref):
    hi = lax.Precision.HIGHEST
    nt = ((1,), (1,)), ((), ())
    a = a_ref[...]
    x = jnp.dot(a, nt_ref[...], precision=hi)
    x = jnp.maximum(lax.dot_general(x, w0_ref[...], nt, precision=hi), 0.0)
    x = jnp.dot(a, x, precision=hi)
    x = jnp.maximum(lax.dot_general(x, w1_ref[...], nt, precision=hi), 0.0)
    x = jnp.dot(a, x, precision=hi)
    x = jnp.maximum(lax.dot_general(x, w2_ref[...], nt, precision=hi), 0.0)
    out_ref[...] = lax.dot_general(x, wf_ref[...], nt, precision=hi) \
        + bf_ref[...]


@jax.jit
def _gcn(A, node_table, W0, W1, W2, Wf, bf):
    return pl.pallas_call(
        _gcn_body,
        out_shape=jax.ShapeDtypeStruct((N_NODES, EMB), jnp.float32),
    )(A, node_table, W0, W1, W2, Wf, bf.reshape(1, EMB))


def kernel(game_state, A, obj_table, node_table, W0, W1, W2, Wf, bf):
    gs_flat = game_state.reshape(-1)
    tab_flat = obj_table.reshape(-1)
    out_flat = _sc_gather(gs_flat, tab_flat)
    out = out_flat.reshape(B, EMB, H, W)
    node_embeddings = _gcn(A, node_table, W0, W1, W2, Wf, bf)
    return (out, node_embeddings)


# speedup vs baseline: 209.3747x; 5.1600x over previous
import dataclasses

import jax
import jax.numpy as jnp
from jax import lax
from jax.experimental import pallas as pl
from jax.experimental.pallas import tpu as pltpu
from jax.experimental.pallas import tpu_sc as plsc

N_NODES = 16
NUM_TYPES = 1000
EMB = 4
B, H, W = 4096, 32, 32
LANES = 16

NUM_CORES = 2
NUM_SUBCORES = 16
NW = NUM_CORES * NUM_SUBCORES
IDX_PER_W = W * B // 1
ROW_WORDS = W * B
CHUNK_W = 8192
NCHUNK = ROW_WORDS // CHUNK_W
PLANE_WORDS = H * W * B


def _gather_body(gs_hbm, tab_hbm, out_hbm, tab_v, idx_v0, idx_v1,
                 out_v0, out_v1, sem_in0, sem_in1, sem_out0, sem_out1):
    h = lax.axis_index("s") * NUM_CORES + lax.axis_index("c")
    in_base = h * ROW_WORDS
    idx_bufs = (idx_v0, idx_v1)
    out_bufs = (out_v0, out_v1)
    sems_in = (sem_in0, sem_in1)
    sems_out = (sem_out0, sem_out1)

    pltpu.sync_copy(tab_hbm, tab_v)

    def start_in(c, buf):
        off = in_base + c * CHUNK_W
        return pltpu.async_copy(gs_hbm.at[pl.ds(off, CHUNK_W)],
                                idx_bufs[buf], sems_in[buf])

    def start_out(c, buf):
        off = in_base + c * CHUNK_W
        cps = []
        for e in range(EMB):
            cps.append(pltpu.async_copy(
                out_bufs[buf].at[pl.ds(e * CHUNK_W, CHUNK_W)],
                out_hbm.at[pl.ds(e * PLANE_WORDS + off, CHUNK_W)],
                sems_out[buf]))
        return cps

    def compute(buf):
        idxb = idx_bufs[buf]
        outb = out_bufs[buf]

        @plsc.parallel_loop(0, CHUNK_W // LANES, unroll=8)
        def _(j):
            o = j * LANES
            ii = idxb[pl.ds(o, LANES)]
            i4 = ii * 4
            outb[pl.ds(o, LANES)] = plsc.load_gather(tab_v, [i4])
            outb[pl.ds(o + CHUNK_W, LANES)] = plsc.load_gather(
                tab_v, [i4 + 1])
            outb[pl.ds(o + 2 * CHUNK_W, LANES)] = plsc.load_gather(
                tab_v, [i4 + 2])
            outb[pl.ds(o + 3 * CHUNK_W, LANES)] = plsc.load_gather(
                tab_v, [i4 + 3])

    copies_out = [None, None]
    copy_in = [None, None]
    copy_in[0] = start_in(0, 0)
    for c in range(NCHUNK):
        buf = c & 1
        if c + 1 < NCHUNK:
            copy_in[1 - buf] = start_in(c + 1, 1 - buf)
        copy_in[buf].wait()
        if copies_out[buf] is not None:
            for cp in copies_out[buf]:
                cp.wait()
        compute(buf)
        copies_out[buf] = start_out(c, buf)
    for bufcps in copies_out:
        for cp in bufcps:
            cp.wait()


@jax.jit
def _sc_gather(gs_lin, tab_flat):
    mesh = plsc.VectorSubcoreMesh(core_axis_name="c", subcore_axis_name="s")
    cp = pltpu.CompilerParams()
    if "needs_layout_passes" in pltpu.CompilerParams.__dataclass_fields__:
        cp = dataclasses.replace(cp, needs_layout_passes=False)
    k = pl.kernel(
        _gather_body,
        out_type=jax.ShapeDtypeStruct((EMB * H * W * B,), jnp.float32),
        mesh=mesh,
        scratch_types=[
            pltpu.VMEM((NUM_TYPES * EMB,), jnp.float32),
            pltpu.VMEM((CHUNK_W,), jnp.int32),
            pltpu.VMEM((CHUNK_W,), jnp.int32),
            pltpu.VMEM((EMB * CHUNK_W,), jnp.float32),
            pltpu.VMEM((EMB * CHUNK_W,), jnp.float32),
            pltpu.SemaphoreType.DMA,
            pltpu.SemaphoreType.DMA,
            pltpu.SemaphoreType.DMA,
            pltpu.SemaphoreType.DMA,
        ],
        compiler_params=cp,
    )
    return k(gs_lin, tab_flat)


def _gcn_body(a_ref, nt_ref, w0_ref, w1_ref, w2_ref, wf_ref, bf_ref, out_ref):
    hi = lax.Precision.HIGHEST
    nt = ((1,), (1,)), ((), ())
    a = a_ref[...]
    x = jnp.dot(a, nt_ref[...], precision=hi)
    x = jnp.maximum(lax.dot_general(x, w0_ref[...], nt, precision=hi), 0.0)
    x = jnp.dot(a, x, precision=hi)
    x = jnp.maximum(lax.dot_general(x, w1_ref[...], nt, precision=hi), 0.0)
    x = jnp.dot(a, x, precision=hi)
    x = jnp.maximum(lax.dot_general(x, w2_ref[...], nt, precision=hi), 0.0)
    out_ref[...] = lax.dot_general(x, wf_ref[...], nt, precision=hi) \
        + bf_ref[...]


@jax.jit
def _gcn(A, node_table, W0, W1, W2, Wf, bf):
    return pl.pallas_call(
        _gcn_body,
        out_shape=jax.ShapeDtypeStruct((N_NODES, EMB), jnp.float32),
    )(A, node_table, W0, W1, W2, Wf, bf.reshape(1, EMB))


def kernel(game_state, A, obj_table, node_table, W0, W1, W2, Wf, bf):
    gs_lin = (game_state.transpose(1, 2, 0)
              .reshape(H, W // 8, 8, B // 128, 128)
              .transpose(0, 1, 3, 2, 4)
              .reshape(-1))
    tab_flat = obj_table.reshape(-1)
    out_lin = _sc_gather(gs_lin, tab_flat)
    out = (out_lin.reshape(EMB, H, W // 8, B // 128, 8, 128)
           .transpose(3, 5, 0, 1, 2, 4)
           .reshape(B, EMB, H, W))
    node_embeddings = _gcn(A, node_table, W0, W1, W2, Wf, bf)
    return (out, node_embeddings)


# speedup vs baseline: 268.1727x; 1.2808x over previous
import dataclasses

import jax
import jax.numpy as jnp
from jax import lax
from jax.experimental import pallas as pl
from jax.experimental.pallas import tpu as pltpu
from jax.experimental.pallas import tpu_sc as plsc

N_NODES = 16
NUM_TYPES = 1000
EMB = 4
B, H, W = 4096, 32, 32
LANES = 16

NUM_CORES = 2
NUM_SUBCORES = 16
NW = NUM_CORES * NUM_SUBCORES
IDX_PER_W = W * B // 1
ROW_WORDS = W * B
CHUNK_W = 8192
NCHUNK = ROW_WORDS // CHUNK_W
PLANE_WORDS = H * W * B


def _gather_body(gs_hbm, tab_hbm, out_hbm, tab_v, idx_v0, idx_v1,
                 out_v0, out_v1, sem_in0, sem_in1, sem_out0, sem_out1):
    h = lax.axis_index("s") * NUM_CORES + lax.axis_index("c")
    in_base = h * ROW_WORDS
    idx_bufs = (idx_v0, idx_v1)
    out_bufs = (out_v0, out_v1)
    sems_in = (sem_in0, sem_in1)
    sems_out = (sem_out0, sem_out1)

    pltpu.sync_copy(tab_hbm, tab_v)

    def start_in(c, buf):
        off = in_base + c * CHUNK_W
        return pltpu.async_copy(gs_hbm.at[pl.ds(off, CHUNK_W)],
                                idx_bufs[buf], sems_in[buf])

    def start_out(c, buf):
        off = in_base + c * CHUNK_W
        cps = []
        for e in range(EMB):
            cps.append(pltpu.async_copy(
                out_bufs[buf].at[pl.ds(e * CHUNK_W, CHUNK_W)],
                out_hbm.at[pl.ds(e * PLANE_WORDS + off, CHUNK_W)],
                sems_out[buf]))
        return cps

    def compute(buf):
        idxb = idx_bufs[buf]
        outb = out_bufs[buf]

        @plsc.parallel_loop(0, CHUNK_W // LANES, unroll=8)
        def _(j):
            o = j * LANES
            ii = idxb[pl.ds(o, LANES)]
            outb[pl.ds(o, LANES)] = plsc.load_gather(tab_v, [ii])
            outb[pl.ds(o + CHUNK_W, LANES)] = plsc.load_gather(
                tab_v, [ii + 1024])
            outb[pl.ds(o + 2 * CHUNK_W, LANES)] = plsc.load_gather(
                tab_v, [ii + 2048])
            outb[pl.ds(o + 3 * CHUNK_W, LANES)] = plsc.load_gather(
                tab_v, [ii + 3072])

    copies_out = [None, None]
    copy_in = [None, None]
    copy_in[0] = start_in(0, 0)
    for c in range(NCHUNK):
        buf = c & 1
        if c + 1 < NCHUNK:
            copy_in[1 - buf] = start_in(c + 1, 1 - buf)
        copy_in[buf].wait()
        if copies_out[buf] is not None:
            for cp in copies_out[buf]:
                cp.wait()
        compute(buf)
        copies_out[buf] = start_out(c, buf)
    for bufcps in copies_out:
        for cp in bufcps:
            cp.wait()


@jax.jit
def _sc_gather(gs_lin, tab_flat):
    mesh = plsc.VectorSubcoreMesh(core_axis_name="c", subcore_axis_name="s")
    cp = pltpu.CompilerParams()
    if "needs_layout_passes" in pltpu.CompilerParams.__dataclass_fields__:
        cp = dataclasses.replace(cp, needs_layout_passes=False)
    k = pl.kernel(
        _gather_body,
        out_type=jax.ShapeDtypeStruct((EMB * H * W * B,), jnp.float32),
        mesh=mesh,
        scratch_types=[
            pltpu.VMEM((EMB * 1024,), jnp.float32),
            pltpu.VMEM((CHUNK_W,), jnp.int32),
            pltpu.VMEM((CHUNK_W,), jnp.int32),
            pltpu.VMEM((EMB * CHUNK_W,), jnp.float32),
            pltpu.VMEM((EMB * CHUNK_W,), jnp.float32),
            pltpu.SemaphoreType.DMA,
            pltpu.SemaphoreType.DMA,
            pltpu.SemaphoreType.DMA,
            pltpu.SemaphoreType.DMA,
        ],
        compiler_params=cp,
    )
    return k(gs_lin, tab_flat)


def _gcn_body(a_ref, nt_ref, w0_ref, w1_ref, w2_ref, wf_ref, bf_ref, out_ref):
    hi = lax.Precision.HIGHEST
    nt = ((1,), (1,)), ((), ())
    a = a_ref[...]
    x = jnp.dot(a, nt_ref[...], precision=hi)
    x = jnp.maximum(lax.dot_general(x, w0_ref[...], nt, precision=hi), 0.0)
    x = jnp.dot(a, x, precision=hi)
    x = jnp.maximum(lax.dot_general(x, w1_ref[...], nt, precision=hi), 0.0)
    x = jnp.dot(a, x, precision=hi)
    x = jnp.maximum(lax.dot_general(x, w2_ref[...], nt, precision=hi), 0.0)
    out_ref[...] = lax.dot_general(x, wf_ref[...], nt, precision=hi) \
        + bf_ref[...]


@jax.jit
def _gcn(A, node_table, W0, W1, W2, Wf, bf):
    return pl.pallas_call(
        _gcn_body,
        out_shape=jax.ShapeDtypeStruct((N_NODES, EMB), jnp.float32),
    )(A, node_table, W0, W1, W2, Wf, bf.reshape(1, EMB))


def kernel(game_state, A, obj_table, node_table, W0, W1, W2, Wf, bf):
    gs_lin = (game_state.transpose(1, 2, 0)
              .reshape(H, W // 8, 8, B // 128, 128)
              .transpose(0, 1, 3, 2, 4)
              .reshape(-1))
    tab_flat = jnp.pad(obj_table.T, ((0, 0), (0, 1024 - NUM_TYPES))) \
        .reshape(-1)
    out_lin = _sc_gather(gs_lin, tab_flat)
    out = (out_lin.reshape(EMB, H, W // 8, B // 128, 8, 128)
           .transpose(3, 5, 0, 1, 2, 4)
           .reshape(B, EMB, H, W))
    node_embeddings = _gcn(A, node_table, W0, W1, W2, Wf, bf)
    return (out, node_embeddings)


# speedup vs baseline: 286.5181x; 1.0684x over previous
import dataclasses

import jax
import jax.numpy as jnp
from jax import lax
from jax.experimental import pallas as pl
from jax.experimental.pallas import tpu as pltpu
from jax.experimental.pallas import tpu_sc as plsc

N_NODES = 16
NUM_TYPES = 1000
EMB = 4
B, H, W = 4096, 32, 32
LANES = 16

NUM_CORES = 2
NUM_SUBCORES = 16
NW = NUM_CORES * NUM_SUBCORES
IDX_PER_W = W * B // 1
ROW_WORDS = W * B
CHUNK_W = 8192
NCHUNK = ROW_WORDS // CHUNK_W
PLANE_WORDS = H * W * B


def _gather_body(gs_hbm, tab_hbm, out_hbm, tab_v, idx_v0, idx_v1,
                 out_v0, out_v1, sem_in0, sem_in1, sem_out0, sem_out1):
    h = lax.axis_index("s") * NUM_CORES + lax.axis_index("c")
    in_base = h * ROW_WORDS
    idx_bufs = (idx_v0, idx_v1)
    out_bufs = (out_v0, out_v1)
    sems_in = (sem_in0, sem_in1)
    sems_out = (sem_out0, sem_out1)

    pltpu.sync_copy(tab_hbm, tab_v)

    def start_in(c, buf):
        off = in_base + c * CHUNK_W
        pltpu.async_copy(gs_hbm.at[pl.ds(off, CHUNK_W)],
                         idx_bufs[buf], sems_in[buf])

    def wait_in(buf):
        pltpu.make_async_copy(gs_hbm.at[pl.ds(0, CHUNK_W)],
                              idx_bufs[buf], sems_in[buf]).wait()

    def start_out(c, buf):
        off = in_base + c * CHUNK_W
        for e in range(EMB):
            pltpu.async_copy(
                out_bufs[buf].at[pl.ds(e * CHUNK_W, CHUNK_W)],
                out_hbm.at[pl.ds(e * PLANE_WORDS + off, CHUNK_W)],
                sems_out[buf])

    def wait_out(buf):
        pltpu.make_async_copy(out_bufs[buf],
                              out_hbm.at[pl.ds(0, EMB * CHUNK_W)],
                              sems_out[buf]).wait()

    def compute(buf):
        idxb = idx_bufs[buf]
        outb = out_bufs[buf]

        @plsc.parallel_loop(0, CHUNK_W // LANES, unroll=8)
        def _(j):
            o = j * LANES
            ii = idxb[pl.ds(o, LANES)]
            outb[pl.ds(o, LANES)] = plsc.load_gather(tab_v, [ii])
            outb[pl.ds(o + CHUNK_W, LANES)] = plsc.load_gather(
                tab_v, [ii + 1024])
            outb[pl.ds(o + 2 * CHUNK_W, LANES)] = plsc.load_gather(
                tab_v, [ii + 2048])
            outb[pl.ds(o + 3 * CHUNK_W, LANES)] = plsc.load_gather(
                tab_v, [ii + 3072])

    start_in(0, 0)
    start_in(1, 1)

    @pl.loop(0, NCHUNK, step=2)
    def _(c):
        for k in (0, 1):
            cc = c + k

            @pl.when(c >= 2)
            def _():
                wait_out(k)

            wait_in(k)
            compute(k)
            start_out(cc, k)

            @pl.when(cc + 2 < NCHUNK)
            def _():
                start_in(cc + 2, k)

    wait_out(0)
    wait_out(1)


@jax.jit
def _sc_gather(gs_lin, tab_flat):
    mesh = plsc.VectorSubcoreMesh(core_axis_name="c", subcore_axis_name="s")
    cp = pltpu.CompilerParams()
    if "needs_layout_passes" in pltpu.CompilerParams.__dataclass_fields__:
        cp = dataclasses.replace(cp, needs_layout_passes=False)
    k = pl.kernel(
        _gather_body,
        out_type=jax.ShapeDtypeStruct((EMB * H * W * B,), jnp.float32),
        mesh=mesh,
        scratch_types=[
            pltpu.VMEM((EMB * 1024,), jnp.float32),
            pltpu.VMEM((CHUNK_W,), jnp.int32),
            pltpu.VMEM((CHUNK_W,), jnp.int32),
            pltpu.VMEM((EMB * CHUNK_W,), jnp.float32),
            pltpu.VMEM((EMB * CHUNK_W,), jnp.float32),
            pltpu.SemaphoreType.DMA,
            pltpu.SemaphoreType.DMA,
            pltpu.SemaphoreType.DMA,
            pltpu.SemaphoreType.DMA,
        ],
        compiler_params=cp,
    )
    return k(gs_lin, tab_flat)


def _gcn_body(a_ref, nt_ref, w0_ref, w1_ref, w2_ref, wf_ref, bf_ref, out_ref):
    hi = lax.Precision.HIGHEST
    nt = ((1,), (1,)), ((), ())
    a = a_ref[...]
    x = jnp.dot(a, nt_ref[...], precision=hi)
    x = jnp.maximum(lax.dot_general(x, w0_ref[...], nt, precision=hi), 0.0)
    x = jnp.dot(a, x, precision=hi)
    x = jnp.maximum(lax.dot_general(x, w1_ref[...], nt, precision=hi), 0.0)
    x = jnp.dot(a, x, precision=hi)
    x = jnp.maximum(lax.dot_general(x, w2_ref[...], nt, precision=hi), 0.0)
    out_ref[...] = lax.dot_general(x, wf_ref[...], nt, precision=hi) \
        + bf_ref[...]


@jax.jit
def _gcn(A, node_table, W0, W1, W2, Wf, bf):
    return pl.pallas_call(
        _gcn_body,
        out_shape=jax.ShapeDtypeStruct((N_NODES, EMB), jnp.float32),
    )(A, node_table, W0, W1, W2, Wf, bf.reshape(1, EMB))


def kernel(game_state, A, obj_table, node_table, W0, W1, W2, Wf, bf):
    gs_lin = (game_state.transpose(1, 2, 0)
              .reshape(H, W // 8, 8, B // 128, 128)
              .transpose(0, 1, 3, 2, 4)
              .reshape(-1))
    tab_flat = jnp.pad(obj_table.T, ((0, 0), (0, 1024 - NUM_TYPES))) \
        .reshape(-1)
    out_lin = _sc_gather(gs_lin, tab_flat)
    out = (out_lin.reshape(EMB, H, W // 8, B // 128, 8, 128)
           .transpose(3, 5, 0, 1, 2, 4)
           .reshape(B, EMB, H, W))
    node_embeddings = _gcn(A, node_table, W0, W1, W2, Wf, bf)
    return (out, node_embeddings)
